# baseline (speedup 1.0000x reference)
import functools

import jax
import jax.numpy as jnp
from jax import lax
from jax.experimental import pallas as pl
from jax.experimental.pallas import tpu as pltpu
from jax.experimental.pallas import tpu_sc as plsc

INV_SQRT_2 = 0.7071067811865475

N_ATOMS = 10000
N_EDGES = 320000
D = 128
NC = 2
NS = 16
NW = NC * NS
E_PER_W = N_EDGES // NW
CHUNK = 40
NCHUNK = E_PER_W // CHUNK
WB_CHUNK = 80
WB_NCHUNK = N_ATOMS // WB_CHUNK
WB_STEPS = -(-WB_NCHUNK // NS)


def _ssilu(x):
    return jax.nn.silu(x) * (1.0 / 0.6)


_BF_PERM = []
for _g in range(4):
    for _i in range(16):
        _BF_PERM += [32 * _g + _i, 32 * _g + 16 + _i]
_BF_PERM = tuple(_BF_PERM)


def _pre_body(h_ref, w1_ref, w2_ref, o_ref):
    x = h_ref[...]
    y = _ssilu(jnp.dot(x, w1_ref[...], preferred_element_type=jnp.float32))
    y = _ssilu(jnp.dot(y, w2_ref[...], preferred_element_type=jnp.float32))
    o_ref[...] = (x + y) * INV_SQRT_2


def _pre_residual(h, w1, w2):
    blk = 2000
    grid = N_ATOMS // blk
    return pl.pallas_call(
        _pre_body,
        grid=(grid,),
        in_specs=[
            pl.BlockSpec((blk, D), lambda i: (i, 0)),
            pl.BlockSpec((D, D), lambda i: (0, 0)),
            pl.BlockSpec((D, D), lambda i: (0, 0)),
        ],
        out_specs=pl.BlockSpec((blk, D), lambda i: (i, 0)),
        out_shape=jax.ShapeDtypeStruct((N_ATOMS, D), jnp.float32),
    )(h, w1, w2)


def _bf_body(bf_ref, w_ref, o_ref):
    o_ref[...] = jnp.dot(bf_ref[...], w_ref[...],
                         preferred_element_type=jnp.float32).astype(jnp.bfloat16)


def _out_shape_f32(shape):
    return jax.ShapeDtypeStruct(shape, jnp.float32)


def _bf_proj(bf, w_bf):
    blk = 4000
    grid = bf.shape[0] // blk
    return pl.pallas_call(
        _bf_body,
        grid=(grid,),
        in_specs=[
            pl.BlockSpec((blk, 16), lambda i: (i, 0)),
            pl.BlockSpec((16, D), lambda i: (0, 0)),
        ],
        out_specs=pl.BlockSpec((blk, D), lambda i: (i, 0)),
        out_shape=jax.ShapeDtypeStruct((bf.shape[0], D), jnp.bfloat16),
    )(bf, w_bf)


def _make_sc_body(e_per_w, nchunk):
  def _sc_body(hres_hbm, mlpbf_hbm, idxs_hbm, idxt_hbm, out_hbm,
             isv0, isv1, isv2, itv0, itv1, itv2,
             rows0, rows1, rows2, bfc0, bfc1, bfc2,
             bounce_v,
             acc_sh,
             semg0, semg1, semg2, semb0, semb1, semb2,
             sems0, sems1, sems2,
             semis0, semis1, semis2, semit0, semit1, semit2):
    c = lax.axis_index("c")
    s = lax.axis_index("s")
    wid = s * NC + c
    isv = (isv0, isv1, isv2)
    itv = (itv0, itv1, itv2)
    rows = (rows0, rows1, rows2)
    bfc = (bfc0, bfc1, bfc2)
    semg = (semg0, semg1, semg2)
    semb = (semb0, semb1, semb2)
    sems = (sems0, sems1, sems2)
    semis = (semis0, semis1, semis2)
    semit = (semit0, semit1, semit2)

    zf = jnp.zeros((16,), jnp.float32)

    def zrow(r, carry):
        for j in range(D // 16):
            bounce_v[r, pl.ds(j * 16, 16)] = zf
        return carry

    lax.fori_loop(0, WB_CHUNK, zrow, 0)
    for k in range(WB_STEPS):
        wc = s + k * NS

        @pl.when(wc < WB_NCHUNK)
        def _():
            pltpu.sync_copy(bounce_v, acc_sh.at[pl.ds(wc * WB_CHUNK, WB_CHUNK)])

    plsc.subcore_barrier()

    base = wid * e_per_w

    def esl(ci):
        return pl.ds(base + ci * CHUNK, CHUNK)

    def prefetch_idx(ci, p):
        pltpu.async_copy(idxs_hbm.at[esl(ci)], isv[p], semis[p])

    def launch(ci, p, wait_scatter):
        if wait_scatter:
            pltpu.make_async_copy(rows[p], acc_sh.at[itv[p]], sems[p]).wait()
        pltpu.async_copy(idxt_hbm.at[esl(ci)], itv[p], semit[p])
        pltpu.make_async_copy(idxs_hbm.at[esl(ci)], isv[p], semis[p]).wait()
        pltpu.async_copy(hres_hbm.at[isv[p]], rows[p], semg[p])
        pltpu.async_copy(mlpbf_hbm.at[esl(ci)], bfc[p], semb[p])

    def process(ci, p, launch_next, launch_ws=True, pf_idx=True):
        pltpu.make_async_copy(hres_hbm.at[isv[p]], rows[p], semg[p]).wait()
        pltpu.make_async_copy(mlpbf_hbm.at[esl(ci)], bfc[p], semb[p]).wait()
        if pf_idx:
            prefetch_idx(ci + 3, p)

        @plsc.parallel_loop(0, CHUNK, unroll=4)
        def _(r):
            for u in range(D // 32):
                w = bfc[p][r, pl.ds(u * 16, 16)]
                lo = lax.bitcast_convert_type(w << 16, jnp.float32)
                hi = lax.bitcast_convert_type(w & (-65536), jnp.float32)
                sl_lo = pl.ds(u * 32, 16)
                sl_hi = pl.ds(u * 32 + 16, 16)
                rows[p][r, sl_lo] = rows[p][r, sl_lo] * lo
                rows[p][r, sl_hi] = rows[p][r, sl_hi] * hi

        pltpu.make_async_copy(idxt_hbm.at[esl(ci)], itv[p], semit[p]).wait()
        pltpu.async_copy(rows[p], acc_sh.at[itv[p]], sems[p], add=True)
        if launch_next:
            launch(ci + 2, (p + 2) % 3, wait_scatter=launch_ws)

    prefetch_idx(0, 0)
    prefetch_idx(1, 1)
    prefetch_idx(2, 2)
    launch(0, 0, False)
    launch(1, 1, False)
    process(0, 0, launch_next=True, launch_ws=False)
    process(1, 1, launch_next=True)
    process(2, 2, launch_next=True)

    tail = 3 + ((nchunk - 6) % 3)
    nmid = (nchunk - 3 - tail) // 3

    def kbody(k, cr):
        ci = 3 * k + 3
        process(ci, 0, launch_next=True)
        process(ci + 1, 1, launch_next=True)
        process(ci + 2, 2, launch_next=True)
        return cr

    lax.fori_loop(0, nmid, kbody, 0)
    for ci in range(3 + 3 * nmid, nchunk):
        process(ci, ci % 3,
                launch_next=ci + 2 < nchunk,
                pf_idx=ci + 3 < nchunk)
    for p in range(3):
        pltpu.make_async_copy(rows[p], acc_sh.at[itv[p]], sems[p]).wait()
    plsc.subcore_barrier()

    for k in range(WB_STEPS):
        wc = s + k * NS

        @pl.when(wc < WB_NCHUNK)
        def _():
            row0 = wc * WB_CHUNK
            pltpu.sync_copy(acc_sh.at[pl.ds(row0, WB_CHUNK)], bounce_v)
            pltpu.sync_copy(bounce_v, out_hbm.at[c, pl.ds(row0, WB_CHUNK)])

  return _sc_body


def _make_sc_kernel(n_edges):
    e_per_w = n_edges // NW
    nchunk = e_per_w // CHUNK
    assert e_per_w * NW == n_edges and nchunk * CHUNK == e_per_w
    return pl.kernel(
        _make_sc_body(e_per_w, nchunk),
        out_type=jax.ShapeDtypeStruct((NC, N_ATOMS, D), jnp.float32),
        mesh=plsc.VectorSubcoreMesh(core_axis_name="c", subcore_axis_name="s"),
        scratch_types=(
            [pltpu.VMEM((CHUNK,), jnp.int32)] * 6
            + [pltpu.VMEM((CHUNK, D), jnp.float32)] * 3
            + [pltpu.VMEM((CHUNK, D // 2), jnp.int32)] * 3
            + [pltpu.VMEM((WB_CHUNK, D), jnp.float32)]
            + [pltpu.VMEM_SHARED((N_ATOMS, D), jnp.float32)]
            + [pltpu.SemaphoreType.DMA] * 15
        ),
    )


_sc_half = _make_sc_kernel(N_EDGES // 2)


def _post_body(scale_ref, p_ref, q_ref, mw_ref, a1_ref, a2_ref, b1_ref, b2_ref, o_ref):
    x = ((p_ref[0] + p_ref[1]) + (q_ref[0] + q_ref[1])) * scale_ref[0]
    x = _ssilu(jnp.dot(x, mw_ref[...], preferred_element_type=jnp.float32))
    y = _ssilu(jnp.dot(x, a1_ref[...], preferred_element_type=jnp.float32))
    y = _ssilu(jnp.dot(y, a2_ref[...], preferred_element_type=jnp.float32))
    x = (x + y) * INV_SQRT_2
    y = _ssilu(jnp.dot(x, b1_ref[...], preferred_element_type=jnp.float32))
    y = _ssilu(jnp.dot(y, b2_ref[...], preferred_element_type=jnp.float32))
    o_ref[...] = (x + y) * INV_SQRT_2


def _post_mlp(p0, p1, mlpW, r1W1, r1W2, r2W1, r2W2, scale):
    blk = 2000
    grid = N_ATOMS // blk
    pspec = pl.BlockSpec((NC, blk, D), lambda i: (0, i, 0))
    wspec = pl.BlockSpec((D, D), lambda i: (0, 0))
    return pl.pallas_call(
        _post_body,
        grid=(grid,),
        in_specs=[
            pl.BlockSpec(memory_space=pltpu.SMEM),
            pspec, pspec,
            wspec, wspec, wspec, wspec, wspec,
        ],
        out_specs=pl.BlockSpec((blk, D), lambda i: (i, 0)),
        out_shape=jax.ShapeDtypeStruct((N_ATOMS, D), jnp.float32),
    )(scale, p0, p1, mlpW, r1W1, r1W2, r2W1, r2W2)


def kernel(h, bf, idx_s, idx_t, W_bf, preW1, preW2, mlpW, r1W1, r1W2, r2W1, r2W2, scale):
    h_res = _pre_residual(h, preW1, preW2)
    idx_s = idx_s.astype(jnp.int32)
    idx_t = idx_t.astype(jnp.int32)
    e2 = N_EDGES // 2
    w_perm = W_bf[:, list(_BF_PERM)]

    def pack_words(mbf):
        return lax.bitcast_convert_type(
            mbf.reshape(mbf.shape[0], D // 2, 2), jnp.int32)

    mbf0 = pack_words(_bf_proj(bf[:e2], w_perm))
    p0 = _sc_half(h_res, mbf0, idx_s[:e2], idx_t[:e2])
    mbf1 = pack_words(_bf_proj(bf[e2:], w_perm))
    p1 = _sc_half(h_res, mbf1, idx_s[e2:], idx_t[e2:])
    scale_arr = jnp.reshape(scale.astype(jnp.float32), (1,))
    return _post_mlp(p0, p1, mlpW, r1W1, r1W2, r2W1, r2W2, scale_arr)


# speedup vs baseline: 2.8584x; 2.8584x over previous
import functools

import jax
import jax.numpy as jnp
from jax import lax
from jax.experimental import pallas as pl
from jax.experimental.pallas import tpu as pltpu
from jax.experimental.pallas import tpu_sc as plsc

INV_SQRT_2 = 0.7071067811865475

N_ATOMS = 10000
N_EDGES = 320000
D = 128
NC = 2
NS = 16
NW = NC * NS
E_PER_W = N_EDGES // NW
CHUNK = 40
NCHUNK = E_PER_W // CHUNK
WB_CHUNK = 80
WB_NCHUNK = N_ATOMS // WB_CHUNK
WB_STEPS = -(-WB_NCHUNK // NS)


def _ssilu(x):
    return jax.nn.silu(x) * (1.0 / 0.6)


_BF_PERM = tuple(
    [32 * _g + _i for _g in range(4) for _i in range(16)]
    + [32 * _g + 16 + _i for _g in range(4) for _i in range(16)]
)


def _pre_body(h_ref, w1_ref, w2_ref, o_ref):
    x = h_ref[...]
    y = _ssilu(jnp.dot(x, w1_ref[...], preferred_element_type=jnp.float32))
    y = _ssilu(jnp.dot(y, w2_ref[...], preferred_element_type=jnp.float32))
    o_ref[...] = (x + y) * INV_SQRT_2


def _pre_residual(h, w1, w2):
    blk = 2000
    grid = N_ATOMS // blk
    return pl.pallas_call(
        _pre_body,
        grid=(grid,),
        in_specs=[
            pl.BlockSpec((blk, D), lambda i: (i, 0)),
            pl.BlockSpec((D, D), lambda i: (0, 0)),
            pl.BlockSpec((D, D), lambda i: (0, 0)),
        ],
        out_specs=pl.BlockSpec((blk, D), lambda i: (i, 0)),
        out_shape=jax.ShapeDtypeStruct((N_ATOMS, D), jnp.float32),
    )(h, w1, w2)


def _bf_body(bf_ref, w_ref, o_ref):
    m = jnp.dot(bf_ref[...], w_ref[...], preferred_element_type=jnp.float32)
    ra = m[:, : D // 2].astype(jnp.bfloat16).astype(jnp.float32)
    rb = m[:, D // 2:].astype(jnp.bfloat16).astype(jnp.float32)
    wa = lax.bitcast_convert_type(ra, jnp.int32)
    wb = lax.bitcast_convert_type(rb, jnp.int32)
    o_ref[...] = lax.shift_right_logical(wa, 16) | wb


def _out_shape_f32(shape):
    return jax.ShapeDtypeStruct(shape, jnp.float32)


def _bf_proj(bf, w_bf):
    blk = 4000
    grid = bf.shape[0] // blk
    return pl.pallas_call(
        _bf_body,
        grid=(grid,),
        in_specs=[
            pl.BlockSpec((blk, 16), lambda i: (i, 0)),
            pl.BlockSpec((16, D), lambda i: (0, 0)),
        ],
        out_specs=pl.BlockSpec((blk, D // 2), lambda i: (i, 0)),
        out_shape=jax.ShapeDtypeStruct((bf.shape[0], D // 2), jnp.int32),
    )(bf, w_bf)


def _make_sc_body(e_per_w, nchunk):
  def _sc_body(hres_hbm, mlpbf_hbm, idxs_hbm, idxt_hbm, out_hbm,
             isv0, isv1, isv2, itv0, itv1, itv2,
             rows0, rows1, rows2, bfc0, bfc1, bfc2,
             bounce_v,
             acc_sh,
             semg0, semg1, semg2, semb0, semb1, semb2,
             sems0, sems1, sems2,
             semis0, semis1, semis2, semit0, semit1, semit2):
    c = lax.axis_index("c")
    s = lax.axis_index("s")
    wid = s * NC + c
    isv = (isv0, isv1, isv2)
    itv = (itv0, itv1, itv2)
    rows = (rows0, rows1, rows2)
    bfc = (bfc0, bfc1, bfc2)
    semg = (semg0, semg1, semg2)
    semb = (semb0, semb1, semb2)
    sems = (sems0, sems1, sems2)
    semis = (semis0, semis1, semis2)
    semit = (semit0, semit1, semit2)

    zf = jnp.zeros((16,), jnp.float32)

    def zrow(r, carry):
        for j in range(D // 16):
            bounce_v[r, pl.ds(j * 16, 16)] = zf
        return carry

    lax.fori_loop(0, WB_CHUNK, zrow, 0)
    for k in range(WB_STEPS):
        wc = s + k * NS

        @pl.when(wc < WB_NCHUNK)
        def _():
            pltpu.sync_copy(bounce_v, acc_sh.at[pl.ds(wc * WB_CHUNK, WB_CHUNK)])

    plsc.subcore_barrier()

    base = wid * e_per_w

    def esl(ci):
        return pl.ds(base + ci * CHUNK, CHUNK)

    def prefetch_idx(ci, p):
        pltpu.async_copy(idxs_hbm.at[esl(ci)], isv[p], semis[p])

    def launch(ci, p, wait_scatter):
        if wait_scatter:
            pltpu.make_async_copy(rows[p], acc_sh.at[itv[p]], sems[p]).wait()
        pltpu.async_copy(idxt_hbm.at[esl(ci)], itv[p], semit[p])
        pltpu.make_async_copy(idxs_hbm.at[esl(ci)], isv[p], semis[p]).wait()
        pltpu.async_copy(hres_hbm.at[isv[p]], rows[p], semg[p])
        pltpu.async_copy(mlpbf_hbm.at[esl(ci)], bfc[p], semb[p])

    def process(ci, p, launch_next, launch_ws=True, pf_idx=True):
        pltpu.make_async_copy(hres_hbm.at[isv[p]], rows[p], semg[p]).wait()
        pltpu.make_async_copy(mlpbf_hbm.at[esl(ci)], bfc[p], semb[p]).wait()
        if pf_idx:
            prefetch_idx(ci + 3, p)

        @plsc.parallel_loop(0, CHUNK, unroll=4)
        def _(r):
            for u in range(D // 32):
                w = bfc[p][r, pl.ds(u * 16, 16)]
                lo = lax.bitcast_convert_type(w << 16, jnp.float32)
                hi = lax.bitcast_convert_type(w & (-65536), jnp.float32)
                sl_lo = pl.ds(u * 32, 16)
                sl_hi = pl.ds(u * 32 + 16, 16)
                rows[p][r, sl_lo] = rows[p][r, sl_lo] * lo
                rows[p][r, sl_hi] = rows[p][r, sl_hi] * hi

        pltpu.make_async_copy(idxt_hbm.at[esl(ci)], itv[p], semit[p]).wait()
        pltpu.async_copy(rows[p], acc_sh.at[itv[p]], sems[p], add=True)
        if launch_next:
            launch(ci + 2, (p + 2) % 3, wait_scatter=launch_ws)

    prefetch_idx(0, 0)
    prefetch_idx(1, 1)
    prefetch_idx(2, 2)
    launch(0, 0, False)
    launch(1, 1, False)
    process(0, 0, launch_next=True, launch_ws=False)
    process(1, 1, launch_next=True)
    process(2, 2, launch_next=True)

    tail = 3 + ((nchunk - 6) % 3)
    nmid = (nchunk - 3 - tail) // 3

    def kbody(k, cr):
        ci = 3 * k + 3
        process(ci, 0, launch_next=True)
        process(ci + 1, 1, launch_next=True)
        process(ci + 2, 2, launch_next=True)
        return cr

    lax.fori_loop(0, nmid, kbody, 0)
    for ci in range(3 + 3 * nmid, nchunk):
        process(ci, ci % 3,
                launch_next=ci + 2 < nchunk,
                pf_idx=ci + 3 < nchunk)
    for p in range(3):
        pltpu.make_async_copy(rows[p], acc_sh.at[itv[p]], sems[p]).wait()
    plsc.subcore_barrier()

    for k in range(WB_STEPS):
        wc = s + k * NS

        @pl.when(wc < WB_NCHUNK)
        def _():
            row0 = wc * WB_CHUNK
            pltpu.sync_copy(acc_sh.at[pl.ds(row0, WB_CHUNK)], bounce_v)
            pltpu.sync_copy(bounce_v, out_hbm.at[c, pl.ds(row0, WB_CHUNK)])

  return _sc_body


def _make_sc_kernel(n_edges):
    e_per_w = n_edges // NW
    nchunk = e_per_w // CHUNK
    assert e_per_w * NW == n_edges and nchunk * CHUNK == e_per_w
    return pl.kernel(
        _make_sc_body(e_per_w, nchunk),
        out_type=jax.ShapeDtypeStruct((NC, N_ATOMS, D), jnp.float32),
        mesh=plsc.VectorSubcoreMesh(core_axis_name="c", subcore_axis_name="s"),
        scratch_types=(
            [pltpu.VMEM((CHUNK,), jnp.int32)] * 6
            + [pltpu.VMEM((CHUNK, D), jnp.float32)] * 3
            + [pltpu.VMEM((CHUNK, D // 2), jnp.int32)] * 3
            + [pltpu.VMEM((WB_CHUNK, D), jnp.float32)]
            + [pltpu.VMEM_SHARED((N_ATOMS, D), jnp.float32)]
            + [pltpu.SemaphoreType.DMA] * 15
        ),
    )


_sc_half = _make_sc_kernel(N_EDGES // 2)


def _post_body(scale_ref, p_ref, q_ref, mw_ref, a1_ref, a2_ref, b1_ref, b2_ref, o_ref):
    x = ((p_ref[0] + p_ref[1]) + (q_ref[0] + q_ref[1])) * scale_ref[0]
    x = _ssilu(jnp.dot(x, mw_ref[...], preferred_element_type=jnp.float32))
    y = _ssilu(jnp.dot(x, a1_ref[...], preferred_element_type=jnp.float32))
    y = _ssilu(jnp.dot(y, a2_ref[...], preferred_element_type=jnp.float32))
    x = (x + y) * INV_SQRT_2
    y = _ssilu(jnp.dot(x, b1_ref[...], preferred_element_type=jnp.float32))
    y = _ssilu(jnp.dot(y, b2_ref[...], preferred_element_type=jnp.float32))
    o_ref[...] = (x + y) * INV_SQRT_2


def _post_mlp(p0, p1, mlpW, r1W1, r1W2, r2W1, r2W2, scale):
    blk = 2000
    grid = N_ATOMS // blk
    pspec = pl.BlockSpec((NC, blk, D), lambda i: (0, i, 0))
    wspec = pl.BlockSpec((D, D), lambda i: (0, 0))
    return pl.pallas_call(
        _post_body,
        grid=(grid,),
        in_specs=[
            pl.BlockSpec(memory_space=pltpu.SMEM),
            pspec, pspec,
            wspec, wspec, wspec, wspec, wspec,
        ],
        out_specs=pl.BlockSpec((blk, D), lambda i: (i, 0)),
        out_shape=jax.ShapeDtypeStruct((N_ATOMS, D), jnp.float32),
    )(scale, p0, p1, mlpW, r1W1, r1W2, r2W1, r2W2)


def kernel(h, bf, idx_s, idx_t, W_bf, preW1, preW2, mlpW, r1W1, r1W2, r2W1, r2W2, scale):
    h_res = _pre_residual(h, preW1, preW2)
    idx_s = idx_s.astype(jnp.int32)
    idx_t = idx_t.astype(jnp.int32)
    e2 = N_EDGES // 2
    w_perm = W_bf[:, list(_BF_PERM)]

    mbf0 = _bf_proj(bf[:e2], w_perm)
    p0 = _sc_half(h_res, mbf0, idx_s[:e2], idx_t[:e2])
    mbf1 = _bf_proj(bf[e2:], w_perm)
    p1 = _sc_half(h_res, mbf1, idx_s[e2:], idx_t[e2:])
    scale_arr = jnp.reshape(scale.astype(jnp.float32), (1,))
    return _post_mlp(p0, p1, mlpW, r1W1, r1W2, r2W1, r2W2, scale_arr)
